# baseline (speedup 1.0000x reference)
import jax
import jax.numpy as jnp
from jax import lax
from jax.experimental import pallas as pl
from jax.experimental.pallas import tpu as pltpu
from jax.experimental.pallas import tpu_sc as plsc

_NC = 2
_NS = 16
_NW = _NC * _NS
_CHUNK = 400


def _copy_body(u_hbm, i_hbm, ou_hbm, oi_hbm, bufs, rsem, wsem):
    wid = lax.axis_index("s") * _NC + lax.axis_index("c")

    def chunk_row(t):
        return pl.multiple_of((t * _NW + wid) * _CHUNK, 8)

    tasks = []
    for (src, dst) in ((u_hbm, ou_hbm), (i_hbm, oi_hbm)):
        nchunks = src.shape[0] // _CHUNK
        for t in range(nchunks // _NW):
            tasks.append((src, dst, chunk_row(t)))

    reads, writes = [], []
    for k, (src, dst, off) in enumerate(tasks):
        b = k % 2
        reads.append(pltpu.make_async_copy(
            src.at[pl.ds(off, _CHUNK), :], bufs.at[b], rsem.at[b]))
        writes.append(pltpu.make_async_copy(
            bufs.at[b], dst.at[pl.ds(off, _CHUNK), :], wsem.at[b]))
    n = len(tasks)
    reads[0].start()
    for k in range(n):
        reads[k].wait()
        if k + 1 < n:
            if k >= 1:
                writes[k - 1].wait()
            reads[k + 1].start()
        writes[k].start()
    writes[n - 1].wait()
    if n >= 2:
        writes[n - 2].wait()

    for (src, dst) in ((u_hbm, ou_hbm), (i_hbm, oi_hbm)):
        nchunks = src.shape[0] // _CHUNK
        full = nchunks // _NW
        left = nchunks % _NW
        if left:
            @pl.when(wid < left)
            def _():
                off = chunk_row(full)
                r = pltpu.make_async_copy(
                    src.at[pl.ds(off, _CHUNK), :], bufs.at[0], rsem.at[0])
                w = pltpu.make_async_copy(
                    bufs.at[0], dst.at[pl.ds(off, _CHUNK), :], wsem.at[0])
                r.start()
                r.wait()
                w.start()
                w.wait()


def kernel(user_emb, item_emb):
    n_u, d = user_emb.shape
    n_i, _ = item_emb.shape
    mesh = plsc.VectorSubcoreMesh(core_axis_name="c", subcore_axis_name="s",
                                  num_cores=_NC, num_subcores=_NS)
    run = pl.kernel(
        _copy_body,
        out_type=[
            jax.ShapeDtypeStruct((n_u, d), user_emb.dtype),
            jax.ShapeDtypeStruct((n_i, d), item_emb.dtype),
        ],
        mesh=mesh,
        scratch_types=[
            pltpu.VMEM((2, _CHUNK, 64), jnp.float32),
            pltpu.SemaphoreType.DMA((2,)),
            pltpu.SemaphoreType.DMA((2,)),
        ],
    )
    out_u, out_i = run(user_emb, item_emb)
    return (out_u, out_i)
